# Initial kernel scaffold; baseline (speedup 1.0000x reference)
import jax
import jax.numpy as jnp
from jax.experimental import pallas as pl


def kernel(proj_range, unproj_range, proj_argmax, px, py):
    raise NotImplementedError("write your pallas kernel here")


# speedup vs baseline: 6.1544x; 6.1544x over previous
import math

import jax
import jax.numpy as jnp
import numpy as np
from jax import lax
from jax.experimental import pallas as pl
from jax.experimental.pallas import tpu as pltpu
from jax.experimental.pallas import tpu_sc as plsc

_S = 5
_NCLS = 20
_H, _W, _P = 64, 2048, 130000
_PAD = 2
_HP, _WP = _H + 2 * _PAD, _W + 2 * _PAD
_NTAB = _HP * _WP
_CUT = 1.0
_CENTER = (_S * _S - 1) // 2

_NW = 32
_PPW = 4096
_C = 128
_NCHUNK = _PPW // _C
_GROUPS = _C // 16
_P_PAD = _NW * _PPW


def _inv_gauss_weights():
    x = np.arange(_S)
    xg = np.tile(x, (_S, 1))
    yg = xg.T
    mean = (_S - 1) / 2.0
    g = 1.0 / (2.0 * math.pi) * np.exp(-((xg - mean) ** 2 + (yg - mean) ** 2) / 2.0)
    g = (g / g.sum()).astype(np.float32)
    return [float(v) for v in (1.0 - g).reshape(-1)]


_W25 = _inv_gauss_weights()
_OFF = [dy * _WP + dx for dy in range(_S) for dx in range(_S)]


def _body(rtab, ctab, px_h, py_h, ur_h, out_h,
          pxv, pyv, urv, idxv, bufr, bufc, outv, sem):
    cid = lax.axis_index("c")
    sid = lax.axis_index("s")
    wid = sid * 2 + cid
    wbase = wid * _PPW

    def chunk_body(i, carry):
        base = wbase + i * _C
        pltpu.sync_copy(px_h.at[pl.ds(base, _C)], pxv)
        pltpu.sync_copy(py_h.at[pl.ds(base, _C)], pyv)
        pltpu.sync_copy(ur_h.at[pl.ds(base, _C)], urv)

        def idx_body(g, c2):
            p = pxv[pl.ds(g * 16, 16)]
            q = pyv[pl.ds(g * 16, 16)]
            b = q * _WP + p
            for s in range(25):
                idxv[pl.ds(s * _C + g * 16, 16)] = b + _OFF[s]
            return c2

        lax.fori_loop(0, _GROUPS, idx_body, 0, unroll=False)

        copies = []
        for s in range(25):
            sl = pl.ds(s * _C, _C)
            copies.append(pltpu.async_copy(rtab.at[idxv.at[sl]], bufr.at[sl], sem))
            copies.append(pltpu.async_copy(ctab.at[idxv.at[sl]], bufc.at[sl], sem))
        for cp in copies:
            cp.wait()

        def grp_body(g, c2):
            l0 = g * 16
            r = urv[pl.ds(l0, 16)]
            dist = []
            meta = []
            for s in range(25):
                cls = bufc[pl.ds(s * _C + l0, 16)]
                if s == _CENTER:
                    d = jnp.zeros((16,), jnp.float32)
                else:
                    u = bufr[pl.ds(s * _C + l0, 16)]
                    u = jnp.where(u < 0.0, jnp.inf, u)
                    d = jnp.abs(u - r) * _W25[s]
                dist.append(d)
                meta.append(cls * 32 + s)
            votes = []
            for k in range(5):
                bd = dist[0]
                bm = meta[0]
                for j in range(1, 25):
                    m = dist[j] < bd
                    bd = jnp.where(m, dist[j], bd)
                    bm = jnp.where(m, meta[j], bm)
                bidx = jnp.bitwise_and(bm, 31)
                bcls = jnp.right_shift(bm, 5)
                votes.append(jnp.where(bd > _CUT, _NCLS, bcls))
                if k < 4:
                    for j in range(25):
                        dist[j] = jnp.where(bidx == j, jnp.inf, dist[j])
            zero = jnp.zeros((16,), jnp.int32)
            one = jnp.full((16,), 1, jnp.int32)
            best_cnt = zero
            for v in votes:
                best_cnt = best_cnt + jnp.where(v == 1, one, zero)
            best_cls = one
            for c in range(2, _NCLS):
                cnt = zero
                for v in votes:
                    cnt = cnt + jnp.where(v == c, one, zero)
                m = cnt > best_cnt
                best_cnt = jnp.where(m, cnt, best_cnt)
                best_cls = jnp.where(m, jnp.full((16,), c, jnp.int32), best_cls)
            outv[pl.ds(l0, 16)] = best_cls
            return c2

        lax.fori_loop(0, _GROUPS, grp_body, 0, unroll=False)
        pltpu.sync_copy(outv, out_h.at[pl.ds(base, _C)])
        return carry

    lax.fori_loop(0, _NCHUNK, chunk_body, 0, unroll=False)


def kernel(proj_range, unproj_range, proj_argmax, px, py):
    rtab = jnp.pad(proj_range, _PAD).reshape(-1)
    ctab = jnp.pad(proj_argmax, _PAD).reshape(-1)
    pxp = jnp.pad(px, (0, _P_PAD - _P))
    pyp = jnp.pad(py, (0, _P_PAD - _P))
    urp = jnp.pad(unproj_range, (0, _P_PAD - _P))
    mesh = plsc.VectorSubcoreMesh(core_axis_name="c", subcore_axis_name="s")
    f = pl.kernel(
        _body,
        out_type=jax.ShapeDtypeStruct((_P_PAD,), jnp.int32),
        mesh=mesh,
        scratch_types=[
            pltpu.VMEM((_C,), jnp.int32),
            pltpu.VMEM((_C,), jnp.int32),
            pltpu.VMEM((_C,), jnp.float32),
            pltpu.VMEM((25 * _C,), jnp.int32),
            pltpu.VMEM((25 * _C,), jnp.float32),
            pltpu.VMEM((25 * _C,), jnp.int32),
            pltpu.VMEM((_C,), jnp.int32),
            pltpu.SemaphoreType.DMA,
        ],
    )
    out = f(rtab, ctab, pxp, pyp, urp)
    return out[:_P]


# speedup vs baseline: 14.8469x; 2.4124x over previous
import math

import jax
import jax.numpy as jnp
import numpy as np
from jax import lax
from jax.experimental import pallas as pl
from jax.experimental.pallas import tpu as pltpu
from jax.experimental.pallas import tpu_sc as plsc

_S = 5
_NCLS = 20
_H, _W, _P = 64, 2048, 130000
_PAD = 2
_HP, _WP = _H + 2 * _PAD, _W + 2 * _PAD
_NTAB = _HP * _WP
_CUT = 1.0
_CENTER = (_S * _S - 1) // 2

_NW = 32
_PPW = 4096
_C = 512
_NCHUNK = _PPW // _C
_GROUPS = _C // 16
_P_PAD = _NW * _PPW


def _inv_gauss_weights():
    x = np.arange(_S)
    xg = np.tile(x, (_S, 1))
    yg = xg.T
    mean = (_S - 1) / 2.0
    g = 1.0 / (2.0 * math.pi) * np.exp(-((xg - mean) ** 2 + (yg - mean) ** 2) / 2.0)
    g = (g / g.sum()).astype(np.float32)
    return [float(v) for v in (1.0 - g).reshape(-1)]


_W25 = _inv_gauss_weights()
_OFF = [dy * _WP + dx for dy in range(_S) for dx in range(_S)]


def _body(rtab, ctab, px_h, py_h, ur_h, out_h,
          r_sh, c_sh, pxv, pyv, urv, idxv, bufr, bufc, outv, sem):
    cid = lax.axis_index("c")
    sid = lax.axis_index("s")
    wid = sid * 2 + cid
    wbase = wid * _PPW

    @pl.when(sid == 0)
    def _stage_tables():
        pltpu.sync_copy(rtab, r_sh)
        pltpu.sync_copy(ctab, c_sh)

    plsc.subcore_barrier()

    def chunk_body(i, carry):
        base = wbase + i * _C
        pltpu.sync_copy(px_h.at[pl.ds(base, _C)], pxv)
        pltpu.sync_copy(py_h.at[pl.ds(base, _C)], pyv)
        pltpu.sync_copy(ur_h.at[pl.ds(base, _C)], urv)

        def idx_body(g, c2):
            p = pxv[pl.ds(g * 16, 16)]
            q = pyv[pl.ds(g * 16, 16)]
            b = q * _WP + p
            for s in range(25):
                idxv[pl.ds(s * _C + g * 16, 16)] = b + _OFF[s]
            return c2

        lax.fori_loop(0, _GROUPS, idx_body, 0, unroll=False)

        copies = []
        for k in range(25 * _C // 128):
            sl = pl.ds(k * 128, 128)
            copies.append(pltpu.async_copy(r_sh.at[idxv.at[sl]], bufr.at[sl], sem))
            copies.append(pltpu.async_copy(c_sh.at[idxv.at[sl]], bufc.at[sl], sem))
        for cp in copies:
            cp.wait()

        def grp_body(g, c2):
            l0 = g * 16
            r = urv[pl.ds(l0, 16)]
            dist = []
            meta = []
            for s in range(25):
                cls = bufc[pl.ds(s * _C + l0, 16)]
                if s == _CENTER:
                    d = jnp.zeros((16,), jnp.float32)
                else:
                    u = bufr[pl.ds(s * _C + l0, 16)]
                    u = jnp.where(u < 0.0, jnp.inf, u)
                    d = jnp.abs(u - r) * _W25[s]
                dist.append(d)
                meta.append(cls * 32 + s)
            votes = []
            for k in range(5):
                bd = dist[0]
                bm = meta[0]
                for j in range(1, 25):
                    m = dist[j] < bd
                    bd = jnp.where(m, dist[j], bd)
                    bm = jnp.where(m, meta[j], bm)
                bidx = jnp.bitwise_and(bm, 31)
                bcls = jnp.right_shift(bm, 5)
                votes.append(jnp.where(bd > _CUT, _NCLS, bcls))
                if k < 4:
                    for j in range(25):
                        dist[j] = jnp.where(bidx == j, jnp.inf, dist[j])
            zero = jnp.zeros((16,), jnp.int32)
            one = jnp.full((16,), 1, jnp.int32)
            best_cnt = zero
            for v in votes:
                best_cnt = best_cnt + jnp.where(v == 1, one, zero)
            best_cls = one
            for c in range(2, _NCLS):
                cnt = zero
                for v in votes:
                    cnt = cnt + jnp.where(v == c, one, zero)
                m = cnt > best_cnt
                best_cnt = jnp.where(m, cnt, best_cnt)
                best_cls = jnp.where(m, jnp.full((16,), c, jnp.int32), best_cls)
            outv[pl.ds(l0, 16)] = best_cls
            return c2

        lax.fori_loop(0, _GROUPS, grp_body, 0, unroll=False)
        pltpu.sync_copy(outv, out_h.at[pl.ds(base, _C)])
        return carry

    lax.fori_loop(0, _NCHUNK, chunk_body, 0, unroll=False)


def kernel(proj_range, unproj_range, proj_argmax, px, py):
    rtab = jnp.pad(proj_range, _PAD).reshape(-1)
    ctab = jnp.pad(proj_argmax, _PAD).reshape(-1)
    pxp = jnp.pad(px, (0, _P_PAD - _P))
    pyp = jnp.pad(py, (0, _P_PAD - _P))
    urp = jnp.pad(unproj_range, (0, _P_PAD - _P))
    mesh = plsc.VectorSubcoreMesh(core_axis_name="c", subcore_axis_name="s")
    f = pl.kernel(
        _body,
        out_type=jax.ShapeDtypeStruct((_P_PAD,), jnp.int32),
        mesh=mesh,
        scratch_types=[
            pltpu.VMEM_SHARED((_NTAB,), jnp.float32),
            pltpu.VMEM_SHARED((_NTAB,), jnp.int32),
            pltpu.VMEM((_C,), jnp.int32),
            pltpu.VMEM((_C,), jnp.int32),
            pltpu.VMEM((_C,), jnp.float32),
            pltpu.VMEM((25 * _C,), jnp.int32),
            pltpu.VMEM((25 * _C,), jnp.float32),
            pltpu.VMEM((25 * _C,), jnp.int32),
            pltpu.VMEM((_C,), jnp.int32),
            pltpu.SemaphoreType.DMA,
        ],
    )
    out = f(rtab, ctab, pxp, pyp, urp)
    return out[:_P]


# speedup vs baseline: 15.8118x; 1.0650x over previous
import math

import jax
import jax.numpy as jnp
import numpy as np
from jax import lax
from jax.experimental import pallas as pl
from jax.experimental.pallas import tpu as pltpu
from jax.experimental.pallas import tpu_sc as plsc

_S = 5
_NCLS = 20
_H, _W, _P = 64, 2048, 130000
_PAD = 2
_HP, _WP = _H + 2 * _PAD, _W + 2 * _PAD
_NTAB = _HP * _WP
_CUT = 1.0
_CENTER = (_S * _S - 1) // 2

_NW = 32
_PPW = 4096
_C = 512
_NCHUNK = _PPW // _C
_GROUPS = _C // 16
_P_PAD = _NW * _PPW


def _inv_gauss_weights():
    x = np.arange(_S)
    xg = np.tile(x, (_S, 1))
    yg = xg.T
    mean = (_S - 1) / 2.0
    g = 1.0 / (2.0 * math.pi) * np.exp(-((xg - mean) ** 2 + (yg - mean) ** 2) / 2.0)
    g = (g / g.sum()).astype(np.float32)
    return [float(v) for v in (1.0 - g).reshape(-1)]


_W25 = _inv_gauss_weights()
_OFF = [dy * _WP + dx for dy in range(_S) for dx in range(_S)]


def _body(rtab, ctab, px_h, py_h, ur_h, out_h,
          r_sh, c_sh, pxv, pyv,
          urv0, idxv0, bufr0, bufc0, sem0,
          urv1, idxv1, bufr1, bufc1, sem1,
          outv):
    cid = lax.axis_index("c")
    sid = lax.axis_index("s")
    wid = sid * 2 + cid
    wbase = wid * _PPW

    slots = ((urv0, idxv0, bufr0, bufc0, sem0),
             (urv1, idxv1, bufr1, bufc1, sem1))

    @pl.when(sid == 0)
    def _stage_tables():
        pltpu.sync_copy(rtab, r_sh)
        pltpu.sync_copy(ctab, c_sh)

    plsc.subcore_barrier()

    def fire(slot, i):
        urv, idxv, bufr, bufc, sem = slots[slot]
        base = wbase + i * _C
        pltpu.sync_copy(px_h.at[pl.ds(base, _C)], pxv)
        pltpu.sync_copy(py_h.at[pl.ds(base, _C)], pyv)
        pltpu.sync_copy(ur_h.at[pl.ds(base, _C)], urv)

        def idx_body(g, c2):
            p = pxv[pl.ds(g * 16, 16)]
            q = pyv[pl.ds(g * 16, 16)]
            b = q * _WP + p
            for s in range(25):
                idxv[pl.ds(s * _C + g * 16, 16)] = b + _OFF[s]
            return c2

        lax.fori_loop(0, _GROUPS, idx_body, 0, unroll=False)
        for k in range(25 * _C // 128):
            sl = pl.ds(k * 128, 128)
            pltpu.async_copy(r_sh.at[idxv.at[sl]], bufr.at[sl], sem)
            pltpu.async_copy(c_sh.at[idxv.at[sl]], bufc.at[sl], sem)

    def drain(slot):
        _, _, bufr, bufc, sem = slots[slot]
        pltpu.make_async_copy(rtab.at[pl.ds(0, 25 * _C)], bufr, sem).wait()
        pltpu.make_async_copy(ctab.at[pl.ds(0, 25 * _C)], bufc, sem).wait()

    def compute(slot, i):
        urv, _, bufr, bufc, _ = slots[slot]
        base = wbase + i * _C

        def grp_body(g, c2):
            l0 = g * 16
            r = urv[pl.ds(l0, 16)]
            dist = []
            meta = []
            for s in range(25):
                cls = bufc[pl.ds(s * _C + l0, 16)]
                if s == _CENTER:
                    d = jnp.zeros((16,), jnp.float32)
                else:
                    u = bufr[pl.ds(s * _C + l0, 16)]
                    u = jnp.where(u < 0.0, jnp.inf, u)
                    d = jnp.abs(u - r) * _W25[s]
                dist.append(d)
                meta.append(cls * 32 + s)
            votes = []
            for k in range(5):
                bd = dist[0]
                bm = meta[0]
                for j in range(1, 25):
                    m = dist[j] < bd
                    bd = jnp.where(m, dist[j], bd)
                    bm = jnp.where(m, meta[j], bm)
                bidx = jnp.bitwise_and(bm, 31)
                bcls = jnp.right_shift(bm, 5)
                votes.append(jnp.where(bd > _CUT, _NCLS, bcls))
                if k < 4:
                    for j in range(25):
                        dist[j] = jnp.where(bidx == j, jnp.inf, dist[j])
            zero = jnp.zeros((16,), jnp.int32)
            one = jnp.full((16,), 1, jnp.int32)
            best_cnt = zero
            for v in votes:
                best_cnt = best_cnt + jnp.where(v == 1, one, zero)
            best_cls = one
            for c in range(2, _NCLS):
                cnt = zero
                for v in votes:
                    cnt = cnt + jnp.where(v == c, one, zero)
                m = cnt > best_cnt
                best_cnt = jnp.where(m, cnt, best_cnt)
                best_cls = jnp.where(m, jnp.full((16,), c, jnp.int32), best_cls)
            outv[pl.ds(l0, 16)] = best_cls
            return c2

        lax.fori_loop(0, _GROUPS, grp_body, 0, unroll=False)
        pltpu.sync_copy(outv, out_h.at[pl.ds(base, _C)])

    fire(0, 0)

    def pair_body(j, carry):
        i0 = j * 2
        fire(1, i0 + 1)
        drain(0)
        compute(0, i0)

        @pl.when(j < _NCHUNK // 2 - 1)
        def _fire_next():
            fire(0, i0 + 2)

        drain(1)
        compute(1, i0 + 1)
        return carry

    lax.fori_loop(0, _NCHUNK // 2, pair_body, 0, unroll=False)


def kernel(proj_range, unproj_range, proj_argmax, px, py):
    rtab = jnp.pad(proj_range, _PAD).reshape(-1)
    ctab = jnp.pad(proj_argmax, _PAD).reshape(-1)
    pxp = jnp.pad(px, (0, _P_PAD - _P))
    pyp = jnp.pad(py, (0, _P_PAD - _P))
    urp = jnp.pad(unproj_range, (0, _P_PAD - _P))
    mesh = plsc.VectorSubcoreMesh(core_axis_name="c", subcore_axis_name="s")
    f = pl.kernel(
        _body,
        out_type=jax.ShapeDtypeStruct((_P_PAD,), jnp.int32),
        mesh=mesh,
        scratch_types=[
            pltpu.VMEM_SHARED((_NTAB,), jnp.float32),
            pltpu.VMEM_SHARED((_NTAB,), jnp.int32),
            pltpu.VMEM((_C,), jnp.int32),
            pltpu.VMEM((_C,), jnp.int32),
            pltpu.VMEM((_C,), jnp.float32),
            pltpu.VMEM((25 * _C,), jnp.int32),
            pltpu.VMEM((25 * _C,), jnp.float32),
            pltpu.VMEM((25 * _C,), jnp.int32),
            pltpu.SemaphoreType.DMA,
            pltpu.VMEM((_C,), jnp.float32),
            pltpu.VMEM((25 * _C,), jnp.int32),
            pltpu.VMEM((25 * _C,), jnp.float32),
            pltpu.VMEM((25 * _C,), jnp.int32),
            pltpu.SemaphoreType.DMA,
            pltpu.VMEM((_C,), jnp.int32),
        ],
    )
    out = f(rtab, ctab, pxp, pyp, urp)
    return out[:_P]


# speedup vs baseline: 17.6233x; 1.1146x over previous
import math

import jax
import jax.numpy as jnp
import numpy as np
from jax import lax
from jax.experimental import pallas as pl
from jax.experimental.pallas import tpu as pltpu
from jax.experimental.pallas import tpu_sc as plsc

_S = 5
_NCLS = 20
_H, _W, _P = 64, 2048, 130000
_PAD = 2
_HP, _WP = _H + 2 * _PAD, _W + 2 * _PAD
_NTAB = _HP * _WP
_CUT = 1.0
_CENTER = (_S * _S - 1) // 2

_NW = 32
_PPW = 4096
_C = 512
_NCHUNK = _PPW // _C
_GROUPS = _C // 16
_P_PAD = _NW * _PPW


def _inv_gauss_weights():
    x = np.arange(_S)
    xg = np.tile(x, (_S, 1))
    yg = xg.T
    mean = (_S - 1) / 2.0
    g = 1.0 / (2.0 * math.pi) * np.exp(-((xg - mean) ** 2 + (yg - mean) ** 2) / 2.0)
    g = (g / g.sum()).astype(np.float32)
    return [float(v) for v in (1.0 - g).reshape(-1)]


_W25 = _inv_gauss_weights()
_OFF = [dy * _WP + dx for dy in range(_S) for dx in range(_S)]


def _body(rtab, ctab, px_h, py_h, ur_h, out_h,
          r_sh, c_sh, pxv, pyv,
          urv0, idxv0, bufr0, bufc0, sem0,
          urv1, idxv1, bufr1, bufc1, sem1,
          outv):
    cid = lax.axis_index("c")
    sid = lax.axis_index("s")
    wid = sid * 2 + cid
    wbase = wid * _PPW

    slots = ((urv0, idxv0, bufr0, bufc0, sem0),
             (urv1, idxv1, bufr1, bufc1, sem1))

    @pl.when(sid == 0)
    def _stage_tables():
        pltpu.sync_copy(rtab, r_sh)
        pltpu.sync_copy(ctab, c_sh)

    plsc.subcore_barrier()

    def fire(slot, i):
        urv, idxv, bufr, bufc, sem = slots[slot]
        base = wbase + i * _C
        pltpu.sync_copy(px_h.at[pl.ds(base, _C)], pxv)
        pltpu.sync_copy(py_h.at[pl.ds(base, _C)], pyv)
        pltpu.sync_copy(ur_h.at[pl.ds(base, _C)], urv)

        def idx_body(g, c2):
            p = pxv[pl.ds(g * 16, 16)]
            q = pyv[pl.ds(g * 16, 16)]
            b = q * _WP + p
            for s in range(25):
                idxv[pl.ds(s * _C + g * 16, 16)] = b + _OFF[s]
            return c2

        lax.fori_loop(0, _GROUPS, idx_body, 0, unroll=False)
        for k in range(25 * _C // 128):
            sl = pl.ds(k * 128, 128)
            pltpu.async_copy(r_sh.at[idxv.at[sl]], bufr.at[sl], sem)
            pltpu.async_copy(c_sh.at[idxv.at[sl]], bufc.at[sl], sem)

    def drain(slot):
        _, _, bufr, bufc, sem = slots[slot]
        pltpu.make_async_copy(rtab.at[pl.ds(0, 25 * _C)], bufr, sem).wait()
        pltpu.make_async_copy(ctab.at[pl.ds(0, 25 * _C)], bufc, sem).wait()

    def compute(slot, i):
        urv, _, bufr, bufc, _ = slots[slot]
        base = wbase + i * _C

        def grp_body(g, c2):
            l0 = g * 16
            r = urv[pl.ds(l0, 16)]
            dist = []
            meta = []
            for s in range(25):
                cls = bufc[pl.ds(s * _C + l0, 16)]
                if s == _CENTER:
                    d = jnp.zeros((16,), jnp.float32)
                else:
                    u = bufr[pl.ds(s * _C + l0, 16)]
                    d = jnp.abs(u - r) * _W25[s]
                dist.append(d)
                meta.append(cls * 32 + s)
            votes = []
            for k in range(5):
                items = list(zip(dist, meta))
                while len(items) > 1:
                    nxt = []
                    for t in range(0, len(items) - 1, 2):
                        (da, ma), (db, mb) = items[t], items[t + 1]
                        m = db < da
                        nxt.append((jnp.where(m, db, da), jnp.where(m, mb, ma)))
                    if len(items) % 2:
                        nxt.append(items[-1])
                    items = nxt
                bd, bm = items[0]
                votes.append(jnp.where(bd > _CUT, _NCLS, jnp.right_shift(bm, 5)))
                if k < 4:
                    for j in range(25):
                        dist[j] = jnp.where(bm == meta[j], jnp.inf, dist[j])
            keys = []
            for k in range(5):
                vk = votes[k]
                cnt = jnp.full((16,), 0, jnp.int32)
                for j in range(5):
                    cnt = cnt + jnp.where(vk == votes[j], 1, 0)
                ex = (vk == 0) | (vk == _NCLS)
                keys.append(jnp.where(ex, 30, cnt * 32 + (31 - vk)))
            best = keys[0]
            for k in range(1, 5):
                best = jnp.maximum(best, keys[k])
            outv[pl.ds(l0, 16)] = 31 - jnp.bitwise_and(best, 31)
            return c2

        lax.fori_loop(0, _GROUPS, grp_body, 0, unroll=False)
        pltpu.sync_copy(outv, out_h.at[pl.ds(base, _C)])

    fire(0, 0)

    def pair_body(j, carry):
        i0 = j * 2
        fire(1, i0 + 1)
        drain(0)
        compute(0, i0)

        @pl.when(j < _NCHUNK // 2 - 1)
        def _fire_next():
            fire(0, i0 + 2)

        drain(1)
        compute(1, i0 + 1)
        return carry

    lax.fori_loop(0, _NCHUNK // 2, pair_body, 0, unroll=False)


def kernel(proj_range, unproj_range, proj_argmax, px, py):
    rtab = jnp.pad(proj_range, _PAD).reshape(-1)
    ctab = jnp.pad(proj_argmax, _PAD).reshape(-1)
    pxp = jnp.pad(px, (0, _P_PAD - _P))
    pyp = jnp.pad(py, (0, _P_PAD - _P))
    urp = jnp.pad(unproj_range, (0, _P_PAD - _P))
    mesh = plsc.VectorSubcoreMesh(core_axis_name="c", subcore_axis_name="s")
    f = pl.kernel(
        _body,
        out_type=jax.ShapeDtypeStruct((_P_PAD,), jnp.int32),
        mesh=mesh,
        scratch_types=[
            pltpu.VMEM_SHARED((_NTAB,), jnp.float32),
            pltpu.VMEM_SHARED((_NTAB,), jnp.int32),
            pltpu.VMEM((_C,), jnp.int32),
            pltpu.VMEM((_C,), jnp.int32),
            pltpu.VMEM((_C,), jnp.float32),
            pltpu.VMEM((25 * _C,), jnp.int32),
            pltpu.VMEM((25 * _C,), jnp.float32),
            pltpu.VMEM((25 * _C,), jnp.int32),
            pltpu.SemaphoreType.DMA,
            pltpu.VMEM((_C,), jnp.float32),
            pltpu.VMEM((25 * _C,), jnp.int32),
            pltpu.VMEM((25 * _C,), jnp.float32),
            pltpu.VMEM((25 * _C,), jnp.int32),
            pltpu.SemaphoreType.DMA,
            pltpu.VMEM((_C,), jnp.int32),
        ],
    )
    out = f(rtab, ctab, pxp, pyp, urp)
    return out[:_P]


# speedup vs baseline: 18.0206x; 1.0225x over previous
import math

import jax
import jax.numpy as jnp
import numpy as np
from jax import lax
from jax.experimental import pallas as pl
from jax.experimental.pallas import tpu as pltpu
from jax.experimental.pallas import tpu_sc as plsc

_S = 5
_NCLS = 20
_H, _W, _P = 64, 2048, 130000
_PAD = 2
_HP, _WP = _H + 2 * _PAD, _W + 2 * _PAD
_NTAB = _HP * _WP
_CUT = 1.0
_CENTER = (_S * _S - 1) // 2

_NW = 32
_PPW = 4096
_C = 512
_NCHUNK = _PPW // _C
_GROUPS = _C // 16
_P_PAD = _NW * _PPW


def _inv_gauss_weights():
    x = np.arange(_S)
    xg = np.tile(x, (_S, 1))
    yg = xg.T
    mean = (_S - 1) / 2.0
    g = 1.0 / (2.0 * math.pi) * np.exp(-((xg - mean) ** 2 + (yg - mean) ** 2) / 2.0)
    g = (g / g.sum()).astype(np.float32)
    return [float(v) for v in (1.0 - g).reshape(-1)]


_W25 = _inv_gauss_weights()
_OFF = [dy * _WP + dx for dy in range(_S) for dx in range(_S)]
_SORT5 = [(0, 3), (1, 4), (0, 2), (1, 3), (0, 1), (2, 4), (1, 2), (3, 4), (2, 3)]


def _body(rtab, ctab, px_h, py_h, ur_h, out_h,
          r_sh, c_sh, pxv, pyv,
          urv0, idxv0, bufr0, bufc0, sem0,
          urv1, idxv1, bufr1, bufc1, sem1,
          outv):
    cid = lax.axis_index("c")
    sid = lax.axis_index("s")
    wid = sid * 2 + cid
    wbase = wid * _PPW

    slots = ((urv0, idxv0, bufr0, bufc0, sem0),
             (urv1, idxv1, bufr1, bufc1, sem1))

    @pl.when(sid == 0)
    def _stage_tables():
        pltpu.sync_copy(rtab, r_sh)
        pltpu.sync_copy(ctab, c_sh)

    plsc.subcore_barrier()

    def fire(slot, i):
        urv, idxv, bufr, bufc, sem = slots[slot]
        base = wbase + i * _C
        pltpu.sync_copy(px_h.at[pl.ds(base, _C)], pxv)
        pltpu.sync_copy(py_h.at[pl.ds(base, _C)], pyv)
        pltpu.sync_copy(ur_h.at[pl.ds(base, _C)], urv)

        def idx_body(g, c2):
            p = pxv[pl.ds(g * 16, 16)]
            q = pyv[pl.ds(g * 16, 16)]
            b = q * _WP + p
            for s in range(25):
                idxv[pl.ds(s * _C + g * 16, 16)] = b + _OFF[s]
            return c2

        lax.fori_loop(0, _GROUPS, idx_body, 0, unroll=False)
        for k in range(25 * _C // 128):
            sl = pl.ds(k * 128, 128)
            pltpu.async_copy(r_sh.at[idxv.at[sl]], bufr.at[sl], sem)
            pltpu.async_copy(c_sh.at[idxv.at[sl]], bufc.at[sl], sem)

    def drain(slot):
        _, _, bufr, bufc, sem = slots[slot]
        pltpu.make_async_copy(rtab.at[pl.ds(0, 25 * _C)], bufr, sem).wait()
        pltpu.make_async_copy(ctab.at[pl.ds(0, 25 * _C)], bufc, sem).wait()

    def compute(slot, i):
        urv, _, bufr, bufc, _ = slots[slot]
        base = wbase + i * _C
        lane = lax.iota(jnp.int32, 16)

        def grp_body(g, c2):
            l0 = g * 16
            r = urv[pl.ds(l0, 16)]
            lanes = lane + l0
            cand = []
            for s in range(25):
                if s == _CENTER:
                    kb = jnp.full((16,), 0, jnp.int32)
                else:
                    u = bufr[pl.ds(s * _C + l0, 16)]
                    d = jnp.abs(u - r) * _W25[s]
                    kb = jnp.bitwise_and(lax.bitcast_convert_type(d, jnp.int32), -32)
                cand.append(jnp.bitwise_or(kb, s))

            def ce(v, a, b):
                lo = jnp.minimum(v[a], v[b])
                v[b] = jnp.maximum(v[a], v[b])
                v[a] = lo

            cols = []
            for c in range(5):
                col = cand[c * 5:(c + 1) * 5]
                for a, b in _SORT5:
                    ce(col, a, b)
                cols.append(col)
            S = cols[0]
            for c in range(1, 5):
                lows = [jnp.minimum(S[t], cols[c][4 - t]) for t in range(5)]
                if c < 4:
                    for a, b in _SORT5:
                        ce(lows, a, b)
                S = lows
            clsv = [bufc[pl.ds(s * _C + l0, 16)] for s in range(25)]
            votes = []
            for w in S:
                sidx = jnp.bitwise_and(w, 31)
                cls = clsv[0]
                for s in range(1, 25):
                    cls = jnp.where(sidx == s, clsv[s], cls)
                d = lax.bitcast_convert_type(jnp.bitwise_and(w, -32), jnp.float32)
                votes.append(jnp.where(d > _CUT, _NCLS, cls))
            keys = []
            for k in range(5):
                vk = votes[k]
                cnt = jnp.full((16,), 0, jnp.int32)
                for j in range(5):
                    cnt = cnt + jnp.where(vk == votes[j], 1, 0)
                ex = (vk == 0) | (vk == _NCLS)
                keys.append(jnp.where(ex, 30, cnt * 32 + (31 - vk)))
            best = keys[0]
            for k in range(1, 5):
                best = jnp.maximum(best, keys[k])
            outv[pl.ds(l0, 16)] = 31 - jnp.bitwise_and(best, 31)
            return c2

        lax.fori_loop(0, _GROUPS, grp_body, 0, unroll=False)
        pltpu.sync_copy(outv, out_h.at[pl.ds(base, _C)])

    fire(0, 0)

    def pair_body(j, carry):
        i0 = j * 2
        fire(1, i0 + 1)
        drain(0)
        compute(0, i0)

        @pl.when(j < _NCHUNK // 2 - 1)
        def _fire_next():
            fire(0, i0 + 2)

        drain(1)
        compute(1, i0 + 1)
        return carry

    lax.fori_loop(0, _NCHUNK // 2, pair_body, 0, unroll=False)


def kernel(proj_range, unproj_range, proj_argmax, px, py):
    rtab = jnp.pad(proj_range, _PAD).reshape(-1)
    ctab = jnp.pad(proj_argmax, _PAD).reshape(-1)
    pxp = jnp.pad(px, (0, _P_PAD - _P))
    pyp = jnp.pad(py, (0, _P_PAD - _P))
    urp = jnp.pad(unproj_range, (0, _P_PAD - _P))
    mesh = plsc.VectorSubcoreMesh(core_axis_name="c", subcore_axis_name="s")
    f = pl.kernel(
        _body,
        out_type=jax.ShapeDtypeStruct((_P_PAD,), jnp.int32),
        mesh=mesh,
        scratch_types=[
            pltpu.VMEM_SHARED((_NTAB,), jnp.float32),
            pltpu.VMEM_SHARED((_NTAB,), jnp.int32),
            pltpu.VMEM((_C,), jnp.int32),
            pltpu.VMEM((_C,), jnp.int32),
            pltpu.VMEM((_C,), jnp.float32),
            pltpu.VMEM((25 * _C,), jnp.int32),
            pltpu.VMEM((25 * _C,), jnp.float32),
            pltpu.VMEM((25 * _C,), jnp.int32),
            pltpu.SemaphoreType.DMA,
            pltpu.VMEM((_C,), jnp.float32),
            pltpu.VMEM((25 * _C,), jnp.int32),
            pltpu.VMEM((25 * _C,), jnp.float32),
            pltpu.VMEM((25 * _C,), jnp.int32),
            pltpu.SemaphoreType.DMA,
            pltpu.VMEM((_C,), jnp.int32),
        ],
    )
    out = f(rtab, ctab, pxp, pyp, urp)
    return out[:_P]


# speedup vs baseline: 26.3051x; 1.4597x over previous
import math

import jax
import jax.numpy as jnp
import numpy as np
from jax import lax
from jax.experimental import pallas as pl
from jax.experimental.pallas import tpu as pltpu
from jax.experimental.pallas import tpu_sc as plsc

_S = 5
_NCLS = 20
_H, _W, _P = 64, 2048, 130000
_PAD = 2
_HP, _WP = _H + 2 * _PAD, _W + 2 * _PAD
_NTAB = _HP * _WP
_CUT = 1.0
_CENTER = (_S * _S - 1) // 2

_NW = 32
_PPW = 4096
_C = 512
_NCHUNK = _PPW // _C
_GROUPS = _C // 16
_P_PAD = _NW * _PPW


def _inv_gauss_weights():
    x = np.arange(_S)
    xg = np.tile(x, (_S, 1))
    yg = xg.T
    mean = (_S - 1) / 2.0
    g = 1.0 / (2.0 * math.pi) * np.exp(-((xg - mean) ** 2 + (yg - mean) ** 2) / 2.0)
    g = (g / g.sum()).astype(np.float32)
    return [float(v) for v in (1.0 - g).reshape(-1)]


_W25 = _inv_gauss_weights()
_OFF = [dy * _WP + dx for dy in range(_S) for dx in range(_S)]
_SORT5 = [(0, 3), (1, 4), (0, 2), (1, 3), (0, 1), (2, 4), (1, 2), (3, 4), (2, 3)]


def _body(ttab, px_h, py_h, ur_h, out_h,
          t_sh, pxv, pyv,
          urv0, idxv0, bufw0, sem0,
          urv1, idxv1, bufw1, sem1,
          outv):
    cid = lax.axis_index("c")
    sid = lax.axis_index("s")
    wid = sid * 2 + cid
    wbase = wid * _PPW

    slots = ((urv0, idxv0, bufw0, sem0),
             (urv1, idxv1, bufw1, sem1))

    @pl.when(sid == 0)
    def _stage_tables():
        pltpu.sync_copy(ttab, t_sh)

    plsc.subcore_barrier()

    def fire(slot, i):
        urv, idxv, bufw, sem = slots[slot]
        base = wbase + i * _C
        pltpu.sync_copy(px_h.at[pl.ds(base, _C)], pxv)
        pltpu.sync_copy(py_h.at[pl.ds(base, _C)], pyv)
        pltpu.sync_copy(ur_h.at[pl.ds(base, _C)], urv)

        def idx_body(g, c2):
            p = pxv[pl.ds(g * 16, 16)]
            q = pyv[pl.ds(g * 16, 16)]
            b = q * _WP + p
            for s in range(25):
                idxv[pl.ds(s * _C + g * 16, 16)] = b + _OFF[s]
            return c2

        lax.fori_loop(0, _GROUPS, idx_body, 0, unroll=False)
        for k in range(25 * _C // 128):
            sl = pl.ds(k * 128, 128)
            pltpu.async_copy(t_sh.at[idxv.at[sl]], bufw.at[sl], sem)

    def drain(slot):
        _, _, bufw, sem = slots[slot]
        pltpu.make_async_copy(ttab.at[pl.ds(0, 25 * _C)], bufw, sem).wait()

    def compute(slot, i):
        urv, _, bufw, _ = slots[slot]
        base = wbase + i * _C
        lane = lax.iota(jnp.int32, 16)

        def grp_body(g, c2):
            l0 = g * 16
            r = urv[pl.ds(l0, 16)]
            lanes = lane + l0
            words = [bufw[pl.ds(s * _C + l0, 16)] for s in range(25)]
            cand = []
            for s in range(25):
                if s == _CENTER:
                    kb = jnp.full((16,), 0, jnp.int32)
                else:
                    u = lax.bitcast_convert_type(
                        jnp.bitwise_and(words[s], -32), jnp.float32)
                    d = jnp.abs(u - r) * _W25[s]
                    kb = jnp.bitwise_and(lax.bitcast_convert_type(d, jnp.int32), -32)
                cand.append(jnp.bitwise_or(kb, s))

            def ce(v, a, b):
                lo = jnp.minimum(v[a], v[b])
                v[b] = jnp.maximum(v[a], v[b])
                v[a] = lo

            cols = []
            for c in range(5):
                col = cand[c * 5:(c + 1) * 5]
                for a, b in _SORT5:
                    ce(col, a, b)
                cols.append(col)
            S = cols[0]
            for c in range(1, 5):
                lows = [jnp.minimum(S[t], cols[c][4 - t]) for t in range(5)]
                if c < 4:
                    for a, b in _SORT5:
                        ce(lows, a, b)
                S = lows
            votes = []
            for w in S:
                sidx = jnp.bitwise_and(w, 31)
                wsel = words[0]
                for s in range(1, 25):
                    wsel = jnp.where(sidx == s, words[s], wsel)
                cls = jnp.bitwise_and(wsel, 31)
                d = lax.bitcast_convert_type(jnp.bitwise_and(w, -32), jnp.float32)
                votes.append(jnp.where(d > _CUT, _NCLS, cls))
            keys = []
            for k in range(5):
                vk = votes[k]
                cnt = jnp.full((16,), 0, jnp.int32)
                for j in range(5):
                    cnt = cnt + jnp.where(vk == votes[j], 1, 0)
                ex = (vk == 0) | (vk == _NCLS)
                keys.append(jnp.where(ex, 30, cnt * 32 + (31 - vk)))
            best = keys[0]
            for k in range(1, 5):
                best = jnp.maximum(best, keys[k])
            outv[pl.ds(l0, 16)] = 31 - jnp.bitwise_and(best, 31)
            return c2

        lax.fori_loop(0, _GROUPS, grp_body, 0, unroll=False)
        pltpu.sync_copy(outv, out_h.at[pl.ds(base, _C)])

    fire(0, 0)

    def pair_body(j, carry):
        i0 = j * 2
        fire(1, i0 + 1)
        drain(0)
        compute(0, i0)

        @pl.when(j < _NCHUNK // 2 - 1)
        def _fire_next():
            fire(0, i0 + 2)

        drain(1)
        compute(1, i0 + 1)
        return carry

    lax.fori_loop(0, _NCHUNK // 2, pair_body, 0, unroll=False)


def kernel(proj_range, unproj_range, proj_argmax, px, py):
    rbits = lax.bitcast_convert_type(jnp.pad(proj_range, _PAD), jnp.int32)
    ttab = (jnp.bitwise_and(rbits, -32) | jnp.pad(proj_argmax, _PAD)).reshape(-1)
    pxp = jnp.pad(px, (0, _P_PAD - _P))
    pyp = jnp.pad(py, (0, _P_PAD - _P))
    urp = jnp.pad(unproj_range, (0, _P_PAD - _P))
    mesh = plsc.VectorSubcoreMesh(core_axis_name="c", subcore_axis_name="s")
    f = pl.kernel(
        _body,
        out_type=jax.ShapeDtypeStruct((_P_PAD,), jnp.int32),
        mesh=mesh,
        scratch_types=[
            pltpu.VMEM_SHARED((_NTAB,), jnp.int32),
            pltpu.VMEM((_C,), jnp.int32),
            pltpu.VMEM((_C,), jnp.int32),
            pltpu.VMEM((_C,), jnp.float32),
            pltpu.VMEM((25 * _C,), jnp.int32),
            pltpu.VMEM((25 * _C,), jnp.int32),
            pltpu.SemaphoreType.DMA,
            pltpu.VMEM((_C,), jnp.float32),
            pltpu.VMEM((25 * _C,), jnp.int32),
            pltpu.VMEM((25 * _C,), jnp.int32),
            pltpu.SemaphoreType.DMA,
            pltpu.VMEM((_C,), jnp.int32),
        ],
    )
    out = f(ttab, pxp, pyp, urp)
    return out[:_P]


# speedup vs baseline: 30.3772x; 1.1548x over previous
import math

import jax
import jax.numpy as jnp
import numpy as np
from jax import lax
from jax.experimental import pallas as pl
from jax.experimental.pallas import tpu as pltpu
from jax.experimental.pallas import tpu_sc as plsc

_S = 5
_NCLS = 20
_H, _W, _P = 64, 2048, 130000
_PAD = 2
_HP, _WP = _H + 2 * _PAD, _W + 2 * _PAD
_NTAB = _HP * _WP
_CUT = 1.0
_CENTER = (_S * _S - 1) // 2

_NW = 32
_PPW = 4096
_C = 1024
_NCHUNK = _PPW // _C
_GROUPS = _C // 16
_P_PAD = _NW * _PPW


def _inv_gauss_weights():
    x = np.arange(_S)
    xg = np.tile(x, (_S, 1))
    yg = xg.T
    mean = (_S - 1) / 2.0
    g = 1.0 / (2.0 * math.pi) * np.exp(-((xg - mean) ** 2 + (yg - mean) ** 2) / 2.0)
    g = (g / g.sum()).astype(np.float32)
    return [float(v) for v in (1.0 - g).reshape(-1)]


_W25 = _inv_gauss_weights()
_OFF = [dy * _WP + dx for dy in range(_S) for dx in range(_S)]
_SORT5 = [(0, 3), (1, 4), (0, 2), (1, 3), (0, 1), (2, 4), (1, 2), (3, 4), (2, 3)]


def _body(ttab, px_h, py_h, ur_h, out_h,
          t_sh, pxv, pyv,
          urv0, idxv0, bufw0, sem0,
          urv1, idxv1, bufw1, sem1,
          outv):
    cid = lax.axis_index("c")
    sid = lax.axis_index("s")
    wid = sid * 2 + cid
    wbase = wid * _PPW

    slots = ((urv0, idxv0, bufw0, sem0),
             (urv1, idxv1, bufw1, sem1))

    @pl.when(sid == 0)
    def _stage_tables():
        pltpu.sync_copy(ttab, t_sh)

    plsc.subcore_barrier()

    def fire(slot, i):
        urv, idxv, bufw, sem = slots[slot]
        base = wbase + i * _C
        pltpu.sync_copy(px_h.at[pl.ds(base, _C)], pxv)
        pltpu.sync_copy(py_h.at[pl.ds(base, _C)], pyv)
        pltpu.sync_copy(ur_h.at[pl.ds(base, _C)], urv)

        def idx_body(g, c2):
            p = pxv[pl.ds(g * 16, 16)]
            q = pyv[pl.ds(g * 16, 16)]
            b = q * _WP + p
            for s in range(25):
                idxv[pl.ds(s * _C + g * 16, 16)] = b + _OFF[s]
            return c2

        lax.fori_loop(0, _GROUPS, idx_body, 0, unroll=False)
        for k in range(25):
            sl = pl.ds(k * _C, _C)
            pltpu.async_copy(t_sh.at[idxv.at[sl]], bufw.at[sl], sem)

    def drain(slot):
        _, _, bufw, sem = slots[slot]
        pltpu.make_async_copy(ttab.at[pl.ds(0, 25 * _C)], bufw, sem).wait()

    def compute(slot, i):
        urv, _, bufw, _ = slots[slot]
        base = wbase + i * _C
        lane = lax.iota(jnp.int32, 16)

        def grp_body(g, c2):
            l0 = g * 16
            r = urv[pl.ds(l0, 16)]
            lanes = lane + l0
            words = [bufw[pl.ds(s * _C + l0, 16)] for s in range(25)]
            cand = []
            for s in range(25):
                if s == _CENTER:
                    kb = jnp.full((16,), 0, jnp.int32)
                else:
                    u = lax.bitcast_convert_type(
                        jnp.bitwise_and(words[s], -32), jnp.float32)
                    d = jnp.abs(u - r) * _W25[s]
                    kb = jnp.bitwise_and(lax.bitcast_convert_type(d, jnp.int32), -32)
                cand.append(jnp.bitwise_or(kb, s))

            def ce(v, a, b):
                lo = jnp.minimum(v[a], v[b])
                v[b] = jnp.maximum(v[a], v[b])
                v[a] = lo

            cols = []
            for c in range(5):
                col = cand[c * 5:(c + 1) * 5]
                for a, b in _SORT5:
                    ce(col, a, b)
                cols.append(col)
            S = cols[0]
            for c in range(1, 5):
                lows = [jnp.minimum(S[t], cols[c][4 - t]) for t in range(5)]
                if c < 4:
                    for a, b in _SORT5:
                        ce(lows, a, b)
                S = lows
            votes = []
            for w in S:
                sidx = jnp.bitwise_and(w, 31)
                wsel = words[0]
                for s in range(1, 25):
                    wsel = jnp.where(sidx == s, words[s], wsel)
                cls = jnp.bitwise_and(wsel, 31)
                d = lax.bitcast_convert_type(jnp.bitwise_and(w, -32), jnp.float32)
                votes.append(jnp.where(d > _CUT, _NCLS, cls))
            keys = []
            for k in range(5):
                vk = votes[k]
                cnt = jnp.full((16,), 0, jnp.int32)
                for j in range(5):
                    cnt = cnt + jnp.where(vk == votes[j], 1, 0)
                ex = (vk == 0) | (vk == _NCLS)
                keys.append(jnp.where(ex, 30, cnt * 32 + (31 - vk)))
            best = keys[0]
            for k in range(1, 5):
                best = jnp.maximum(best, keys[k])
            outv[pl.ds(l0, 16)] = 31 - jnp.bitwise_and(best, 31)
            return c2

        lax.fori_loop(0, _GROUPS, grp_body, 0, unroll=False)
        pltpu.sync_copy(outv, out_h.at[pl.ds(base, _C)])

    fire(0, 0)

    def pair_body(j, carry):
        i0 = j * 2
        fire(1, i0 + 1)
        drain(0)
        compute(0, i0)

        @pl.when(j < _NCHUNK // 2 - 1)
        def _fire_next():
            fire(0, i0 + 2)

        drain(1)
        compute(1, i0 + 1)
        return carry

    lax.fori_loop(0, _NCHUNK // 2, pair_body, 0, unroll=False)


def kernel(proj_range, unproj_range, proj_argmax, px, py):
    rbits = lax.bitcast_convert_type(jnp.pad(proj_range, _PAD), jnp.int32)
    ttab = (jnp.bitwise_and(rbits, -32) | jnp.pad(proj_argmax, _PAD)).reshape(-1)
    pxp = jnp.pad(px, (0, _P_PAD - _P))
    pyp = jnp.pad(py, (0, _P_PAD - _P))
    urp = jnp.pad(unproj_range, (0, _P_PAD - _P))
    mesh = plsc.VectorSubcoreMesh(core_axis_name="c", subcore_axis_name="s")
    f = pl.kernel(
        _body,
        out_type=jax.ShapeDtypeStruct((_P_PAD,), jnp.int32),
        mesh=mesh,
        scratch_types=[
            pltpu.VMEM_SHARED((_NTAB,), jnp.int32),
            pltpu.VMEM((_C,), jnp.int32),
            pltpu.VMEM((_C,), jnp.int32),
            pltpu.VMEM((_C,), jnp.float32),
            pltpu.VMEM((25 * _C,), jnp.int32),
            pltpu.VMEM((25 * _C,), jnp.int32),
            pltpu.SemaphoreType.DMA,
            pltpu.VMEM((_C,), jnp.float32),
            pltpu.VMEM((25 * _C,), jnp.int32),
            pltpu.VMEM((25 * _C,), jnp.int32),
            pltpu.SemaphoreType.DMA,
            pltpu.VMEM((_C,), jnp.int32),
        ],
    )
    out = f(ttab, pxp, pyp, urp)
    return out[:_P]
